# baseline (speedup 1.0000x reference)
import functools
import math

import jax
import jax.numpy as jnp
from jax import lax
from jax.experimental import pallas as pl
from jax.experimental.pallas import tpu as pltpu
from jax.experimental.pallas import tpu_sc as plsc

_B, _S, _D, _NQ = 16384, 200, 32, 16
_NC, _NS = 2, 16
_NW = _NC * _NS
_RPW = _B // _NW
_R = 8
_CHUNK = _R * _S
_NCH = _RPW // _R
_SENT = 0x7FFFFFFF


def _sc_encode(ids_flat, mask_flat, table, w_flat, bias):
    mesh = plsc.VectorSubcoreMesh(
        core_axis_name="c", subcore_axis_name="s",
        num_cores=_NC, num_subcores=_NS)

    @functools.partial(
        pl.kernel,
        out_type=jax.ShapeDtypeStruct((_B * _NQ,), jnp.float32),
        mesh=mesh,
        scratch_types=[
            pltpu.VMEM((_CHUNK,), jnp.int32),
            pltpu.VMEM((_CHUNK,), jnp.int32),
            pltpu.VMEM((_CHUNK,), jnp.int32),
            pltpu.VMEM((_CHUNK,), jnp.int32),
            pltpu.VMEM((_CHUNK,), jnp.int32),
            pltpu.VMEM((_CHUNK,), jnp.int32),
            pltpu.VMEM((_CHUNK, _D), jnp.float32),
            pltpu.VMEM((_CHUNK, _D), jnp.float32),
            pltpu.VMEM((_D * _NQ,), jnp.float32),
            pltpu.VMEM((_NQ,), jnp.float32),
            pltpu.VMEM((_R * _NQ,), jnp.float32),
            pltpu.SemaphoreType.DMA,
            pltpu.SemaphoreType.DMA,
        ],
        compiler_params=pltpu.CompilerParams(use_tc_tiling_on_sc=False, needs_layout_passes=False),
    )
    def k(ids_hbm, mask_hbm, table_hbm, w_hbm, b_hbm, out_hbm,
          ids_v0, ids_v1, mask_v0, mask_v1, idx_v0, idx_v1,
          rows_v0, rows_v1, wv, bv, out_v, sem0, sem1):
        wid = lax.axis_index("s") * _NC + lax.axis_index("c")
        tok0 = wid * _RPW * _S
        zero16 = jnp.zeros((16,), jnp.float32)
        lanes = lax.iota(jnp.int32, 16)
        bufs = ((ids_v0, mask_v0, idx_v0, rows_v0, sem0),
                (ids_v1, mask_v1, idx_v1, rows_v1, sem1))

        pltpu.sync_copy(w_hbm, wv)
        pltpu.sync_copy(b_hbm, bv)

        def gather_copy(buf):
            _, _, idx_v, rows_v, sem = buf
            return pltpu.make_async_copy(
                table_hbm.at[plsc.Indices(idx_v, ignored_value=_SENT)],
                rows_v, sem)

        def stage(buf, c):
            ids_v, mask_v, idx_v, rows_v, sem = buf
            off = tok0 + c * _CHUNK
            pltpu.sync_copy(ids_hbm.at[pl.ds(off, _CHUNK)], ids_v)
            pltpu.sync_copy(mask_hbm.at[pl.ds(off, _CHUNK)], mask_v)

            def mul_body(i, carry):
                sl = pl.ds(i * 16, 16)
                idx_v[sl] = jnp.where(mask_v[sl] == 0, _SENT, ids_v[sl])
                return carry

            lax.fori_loop(0, _CHUNK // 16, mul_body, 0)

            def zero_body(i, carry):
                t = i * 4
                for j in range(4):
                    rows_v[t + j, pl.ds(0, 16)] = zero16
                    rows_v[t + j, pl.ds(16, 16)] = zero16
                return carry

            lax.fori_loop(0, _CHUNK // 4, zero_body, 0)
            gather_copy(buf).start()

        def finish(buf, c):
            _, mask_v, _, rows_v, _ = buf
            gather_copy(buf).wait()

            for r in range(_R):
                base = r * _S

                def acc_body(i, carry):
                    a0, a1, b0, b1 = carry
                    t = base + i * 2
                    a0 = a0 + rows_v[t, pl.ds(0, 16)]
                    a1 = a1 + rows_v[t, pl.ds(16, 16)]
                    b0 = b0 + rows_v[t + 1, pl.ds(0, 16)]
                    b1 = b1 + rows_v[t + 1, pl.ds(16, 16)]
                    return (a0, a1, b0, b1)

                a0, a1, b0, b1 = lax.fori_loop(
                    0, _S // 2, acc_body, (zero16, zero16, zero16, zero16))
                s0 = a0 + b0
                s1 = a1 + b1

                cvec = mask_v[pl.ds(base, 16)]
                for j in range(1, 12):
                    cvec = cvec + mask_v[pl.ds(base + j * 16, 16)]
                tail = mask_v[pl.ds(base + 184, 16)]
                cvec = cvec + jnp.where(lanes >= 8, tail, 0)
                cnt = jnp.sum(cvec).astype(jnp.float32)
                den = jnp.broadcast_to(jnp.maximum(cnt, 1.0), (16,))
                p0 = s0 / den
                p1 = s1 / den

                y = bv[pl.ds(0, 16)]
                for kk in range(16):
                    y = y + p0[kk] * wv[pl.ds(kk * 16, 16)]
                for kk in range(16):
                    y = y + p1[kk] * wv[pl.ds((16 + kk) * 16, 16)]

                t = jnp.exp(y * 2.0)
                out_v[pl.ds(r * _NQ, 16)] = (1.0 - 2.0 / (t + 1.0)) * math.pi

            row0 = wid * _RPW + c * _R
            pltpu.sync_copy(out_v, out_hbm.at[pl.ds(row0 * _NQ, _R * _NQ)])

        stage(bufs[0], 0)

        def pair_body(i, carry):
            c = i * 2
            stage(bufs[1], c + 1)
            finish(bufs[0], c)

            @pl.when(i < _NCH // 2 - 1)
            def _():
                stage(bufs[0], c + 2)

            finish(bufs[1], c + 1)
            return carry

        lax.fori_loop(0, _NCH // 2, pair_body, 0)

    return k(ids_flat, mask_flat, table, w_flat, bias)


def kernel(input_ids, attention_mask, emb_table, W, b):
    ids_flat = input_ids.reshape(-1)
    mask_flat = attention_mask.reshape(-1)
    out = _sc_encode(ids_flat, mask_flat, emb_table, W.reshape(-1), b)
    return out.reshape(_B, _NQ)


# speedup vs baseline: 1.1158x; 1.1158x over previous
import functools
import math

import jax
import jax.numpy as jnp
from jax import lax
from jax.experimental import pallas as pl
from jax.experimental.pallas import tpu as pltpu
from jax.experimental.pallas import tpu_sc as plsc

_B, _S, _D, _NQ = 16384, 200, 32, 16
_NC, _NS = 2, 16
_NW = _NC * _NS
_RPW = _B // _NW
_R = 8
_CHUNK = _R * _S
_NCH = _RPW // _R
_SENT = 0x7FFFFFFF


def _sc_sums(idx_flat, table):
    mesh = plsc.VectorSubcoreMesh(
        core_axis_name="c", subcore_axis_name="s",
        num_cores=_NC, num_subcores=_NS)

    @functools.partial(
        pl.kernel,
        out_type=jax.ShapeDtypeStruct((_B * _D,), jnp.float32),
        mesh=mesh,
        scratch_types=[
            pltpu.VMEM((_CHUNK,), jnp.int32),
            pltpu.VMEM((_CHUNK,), jnp.int32),
            pltpu.VMEM((_CHUNK, _D), jnp.float32),
            pltpu.VMEM((_CHUNK, _D), jnp.float32),
            pltpu.VMEM((_R * _D,), jnp.float32),
            pltpu.SemaphoreType.DMA,
            pltpu.SemaphoreType.DMA,
        ],
        compiler_params=pltpu.CompilerParams(use_tc_tiling_on_sc=False),
    )
    def k(idx_hbm, table_hbm, sums_hbm,
          idx_v0, idx_v1, rows_v0, rows_v1, out_v, sem0, sem1):
        wid = lax.axis_index("s") * _NC + lax.axis_index("c")
        tok0 = wid * _RPW * _S
        zero16 = jnp.zeros((16,), jnp.float32)
        bufs = ((idx_v0, rows_v0, sem0),
                (idx_v1, rows_v1, sem1))

        def gather_copy(buf):
            idx_v, rows_v, sem = buf
            return pltpu.make_async_copy(
                table_hbm.at[plsc.Indices(idx_v, ignored_value=_SENT)],
                rows_v, sem)

        def stage(buf, c):
            idx_v, rows_v, sem = buf
            off = tok0 + c * _CHUNK
            pltpu.sync_copy(idx_hbm.at[pl.ds(off, _CHUNK)], idx_v)

            def zero_body(i, carry):
                t = i * 4
                for j in range(4):
                    rows_v[t + j, pl.ds(0, 16)] = zero16
                    rows_v[t + j, pl.ds(16, 16)] = zero16
                return carry

            lax.fori_loop(0, _CHUNK // 4, zero_body, 0)
            gather_copy(buf).start()

        def finish(buf, c):
            _, rows_v, _ = buf
            gather_copy(buf).wait()

            for r in range(_R):
                base = r * _S

                def acc_body(i, carry):
                    a0, a1, b0, b1 = carry
                    t = base + i * 2
                    a0 = a0 + rows_v[t, pl.ds(0, 16)]
                    a1 = a1 + rows_v[t, pl.ds(16, 16)]
                    b0 = b0 + rows_v[t + 1, pl.ds(0, 16)]
                    b1 = b1 + rows_v[t + 1, pl.ds(16, 16)]
                    return (a0, a1, b0, b1)

                a0, a1, b0, b1 = lax.fori_loop(
                    0, _S // 2, acc_body, (zero16, zero16, zero16, zero16))
                out_v[pl.ds(r * _D, 16)] = a0 + b0
                out_v[pl.ds(r * _D + 16, 16)] = a1 + b1

            row0 = wid * _RPW + c * _R
            pltpu.sync_copy(out_v, sums_hbm.at[pl.ds(row0 * _D, _R * _D)])

        stage(bufs[0], 0)

        def pair_body(i, carry):
            c = i * 2
            stage(bufs[1], c + 1)
            finish(bufs[0], c)

            @pl.when(i < _NCH // 2 - 1)
            def _():
                stage(bufs[0], c + 2)

            finish(bufs[1], c + 1)
            return carry

        lax.fori_loop(0, _NCH // 2, pair_body, 0)

    return k(idx_flat, table)


def _tc_finish(mask2d, sums2d, w, bias):
    bm = 1024

    def body(mask_ref, sums_ref, w_ref, b_ref, out_ref):
        cnt = jnp.sum(mask_ref[...].astype(jnp.float32), axis=1, keepdims=True)
        pooled = sums_ref[...] / jnp.maximum(cnt, 1.0)
        y = jnp.dot(pooled, w_ref[...], preferred_element_type=jnp.float32)
        out_ref[...] = jnp.tanh(y + b_ref[...]) * math.pi

    return pl.pallas_call(
        body,
        grid=(_B // bm,),
        in_specs=[
            pl.BlockSpec((bm, _S), lambda i: (i, 0)),
            pl.BlockSpec((bm, _D), lambda i: (i, 0)),
            pl.BlockSpec((_D, _NQ), lambda i: (0, 0)),
            pl.BlockSpec((1, _NQ), lambda i: (0, 0)),
        ],
        out_specs=pl.BlockSpec((bm, _NQ), lambda i: (i, 0)),
        out_shape=jax.ShapeDtypeStruct((_B, _NQ), jnp.float32),
    )(mask2d, sums2d, w, bias.reshape(1, _NQ))


def kernel(input_ids, attention_mask, emb_table, W, b):
    idx_flat = jnp.where(attention_mask == 0, _SENT, input_ids).reshape(-1)
    sums = _sc_sums(idx_flat, emb_table).reshape(_B, _D)
    return _tc_finish(attention_mask, sums, W, b)
